# baseline (speedup 1.0000x reference)
import functools

import jax
import jax.numpy as jnp
from jax import lax
from jax.experimental import pallas as pl
from jax.experimental.pallas import tpu as pltpu
from jax.experimental.pallas import tpu_sc as plsc

N = 50000
E = 800000
D = 128
NB = 16
OUT = 7
EPS = 1e-5

R = 2000
NR = N // R
NCH = 8
CW = D // NCH
NP = N + 48
B = 1600
EB = 51200
EPAD = 16 * EB
NBATCH = EB // B
TROWS = NP // 16
DEGW = 16

def _mesh():
    return plsc.VectorSubcoreMesh(core_axis_name="c", subcore_axis_name="s")


def _sc_degree(dstr, zrows, orows):
    @functools.partial(
        pl.kernel,
        mesh=_mesh(),
        compiler_params=pltpu.CompilerParams(use_tc_tiling_on_sc=False),
        out_type=jax.ShapeDtypeStruct((NP, DEGW), jnp.float32),
        scratch_types=[
            pltpu.VMEM((B,), jnp.int32),
            pltpu.VMEM((B, DEGW), jnp.float32),
            pltpu.VMEM_SHARED((NP, DEGW), jnp.float32),
        ],
    )
    def k(dst_hbm, z_hbm, one_hbm, out_hbm, di, ones_v, buf):
        cid = lax.axis_index("c")
        sid = lax.axis_index("s")

        @pl.when(cid == 0)
        def _():
            pltpu.sync_copy(z_hbm.at[pl.ds(sid * TROWS, TROWS)],
                            buf.at[pl.ds(sid * TROWS, TROWS)])
            pltpu.sync_copy(one_hbm, ones_v)
            plsc.subcore_barrier()

            def body(i, carry):
                e0 = sid * EB + i * B
                pltpu.sync_copy(dst_hbm.at[pl.ds(e0, B)], di)
                pltpu.sync_copy(ones_v, buf.at[di], add=True)
                return carry

            lax.fori_loop(0, NBATCH, body, 0)
            plsc.subcore_barrier()
            pltpu.sync_copy(buf.at[pl.ds(sid * TROWS, TROWS)],
                            out_hbm.at[pl.ds(sid * TROWS, TROWS)])

    return k(dstr, zrows, orows)


def _sc_scatter(y, srcr, dstr):
    @functools.partial(
        pl.kernel,
        mesh=_mesh(),
        compiler_params=pltpu.CompilerParams(use_tc_tiling_on_sc=False),
        out_type=jax.ShapeDtypeStruct((NCH, NP, CW), jnp.float32),
        scratch_types=[
            [pltpu.VMEM((B,), jnp.int32)] * 2,
            [pltpu.VMEM((B,), jnp.int32)] * 2,
            [pltpu.VMEM((B, CW), jnp.float32)] * 2,
            pltpu.VMEM_SHARED((NP, CW), jnp.float32),
            [pltpu.SemaphoreType.DMA] * 2,
            [pltpu.SemaphoreType.DMA] * 2,
            [pltpu.SemaphoreType.DMA] * 2,
        ],
    )
    def k(y_hbm, src_hbm, dst_hbm, out_hbm, si, di, rows, buf, isem, gsem,
          ssem):
        cid = lax.axis_index("c")
        sid = lax.axis_index("s")

        def issue_idx(i, b):
            e0 = sid * EB + i * B
            pltpu.async_copy(src_hbm.at[pl.ds(e0, B)], si[b], isem[b])
            pltpu.async_copy(dst_hbm.at[pl.ds(e0, B)], di[b], isem[b])

        def wait_idx(b):
            pltpu.make_async_copy(src_hbm.at[pl.ds(0, B)], si[b],
                                  isem[b]).wait()
            pltpu.make_async_copy(dst_hbm.at[pl.ds(0, B)], di[b],
                                  isem[b]).wait()

        def wait_scatter(b):
            pltpu.make_async_copy(rows[b], buf.at[di[b]], ssem[b]).wait()

        for kk in range(NCH // 2):
            ch = cid * (NCH // 2) + kk
            pltpu.sync_copy(y_hbm.at[ch].at[pl.ds(sid * TROWS, TROWS)],
                            buf.at[pl.ds(sid * TROWS, TROWS)])
            plsc.subcore_barrier()

            issue_idx(0, 0)

            def body(i2, carry):
                for b in range(2):
                    i = 2 * i2 + b
                    wait_idx(b)
                    pltpu.async_copy(y_hbm.at[ch].at[si[b]], rows[b], gsem[b])

                    @pl.when(i >= 1)
                    def _():
                        wait_scatter(1 - b)

                    @pl.when(i + 1 < NBATCH)
                    def _():
                        issue_idx(i + 1, 1 - b)

                    pltpu.make_async_copy(y_hbm.at[ch].at[si[b]], rows[b],
                                          gsem[b]).wait()
                    pltpu.async_copy(rows[b], buf.at[di[b]], ssem[b],
                                     add=True)
                return carry

            lax.fori_loop(0, NBATCH // 2, body, 0)
            wait_scatter(1)
            plsc.subcore_barrier()
            pltpu.sync_copy(buf.at[pl.ds(sid * TROWS, TROWS)],
                            out_hbm.at[ch].at[pl.ds(sid * TROWS, TROWS)])

    return k(y, srcr, dstr)


def _affine_from_stats(st_ref, g_ref, be_ref):
    m = st_ref[0:1, :] * (1.0 / N)
    var = st_ref[1:2, :] * (1.0 / N) - m * m
    a = g_ref[...] * lax.rsqrt(var + EPS)
    c = be_ref[...] - m * a
    return a, c


def _acc_stats(st_ref, o, first):
    @pl.when(first)
    def _():
        st_ref[...] = jnp.zeros_like(st_ref)
    st_ref[0:1, :] += jnp.sum(o, axis=0, keepdims=True)
    st_ref[1:2, :] += jnp.sum(o * o, axis=0, keepdims=True)


def _mm_stats_body(x_ref, w_ref, b_ref, out_ref, st_ref):
    o = jnp.dot(x_ref[...], w_ref[...],
                preferred_element_type=jnp.float32) + b_ref[...]
    out_ref[...] = o
    _acc_stats(st_ref, o, pl.program_id(0) == 0)


def _mm_stats(x, w, b):
    return pl.pallas_call(
        _mm_stats_body,
        grid=(NR,),
        in_specs=[pl.BlockSpec((R, D), lambda i: (i, 0)),
                  pl.BlockSpec((D, D), lambda i: (0, 0)),
                  pl.BlockSpec((1, D), lambda i: (0, 0))],
        out_specs=[pl.BlockSpec((R, D), lambda i: (i, 0)),
                   pl.BlockSpec((2, D), lambda i: (0, 0))],
        out_shape=[jax.ShapeDtypeStruct((N, D), jnp.float32),
                   jax.ShapeDtypeStruct((2, D), jnp.float32)],
    )(x, w, b)


def _store_chunked(y_ref, yfull):
    for c in range(NCH):
        y_ref[c, :, :] = yfull[:, c * CW:(c + 1) * CW]


def _conv1_body(u_ref, st_ref, g_ref, be_ref, w_ref, dinv_ref, y_ref, h_ref):
    a, c = _affine_from_stats(st_ref, g_ref, be_ref)
    h = jnp.maximum(u_ref[...] * a + c, 0.0)
    h_ref[...] = h
    _store_chunked(y_ref, jnp.dot(h, w_ref[...],
                                  preferred_element_type=jnp.float32)
                   * dinv_ref[...])


def _conv1(u, st, g, be, w, dinv):
    return pl.pallas_call(
        _conv1_body,
        grid=(NR,),
        in_specs=[pl.BlockSpec((R, D), lambda i: (i, 0)),
                  pl.BlockSpec((2, D), lambda i: (0, 0)),
                  pl.BlockSpec((1, D), lambda i: (0, 0)),
                  pl.BlockSpec((1, D), lambda i: (0, 0)),
                  pl.BlockSpec((D, D), lambda i: (0, 0)),
                  pl.BlockSpec((R, 1), lambda i: (i, 0))],
        out_specs=[pl.BlockSpec((NCH, R, CW), lambda i: (0, i, 0)),
                   pl.BlockSpec((R, D), lambda i: (i, 0))],
        out_shape=[jax.ShapeDtypeStruct((NCH, NP, CW), jnp.float32),
                   jax.ShapeDtypeStruct((N, D), jnp.float32)],
    )(u, st, g, be, w, dinv)


def _combine(agg_ref, hp_ref, bc_ref, dinv_ref):
    agg = jnp.concatenate([agg_ref[kk] for kk in range(NCH)], axis=1)
    return jnp.maximum(agg * dinv_ref[...] + bc_ref[...] + hp_ref[...], 0.0)


def _conv23_body(agg_ref, hp_ref, bc_ref, dinv_ref, w_ref, y_ref, h_ref):
    x = _combine(agg_ref, hp_ref, bc_ref, dinv_ref)
    h_ref[...] = x
    _store_chunked(y_ref, jnp.dot(x, w_ref[...],
                                  preferred_element_type=jnp.float32)
                   * dinv_ref[...])


def _conv23(agg, hp, bc, dinv, w):
    return pl.pallas_call(
        _conv23_body,
        grid=(NR,),
        in_specs=[pl.BlockSpec((NCH, R, CW), lambda i: (0, i, 0)),
                  pl.BlockSpec((R, D), lambda i: (i, 0)),
                  pl.BlockSpec((1, D), lambda i: (0, 0)),
                  pl.BlockSpec((R, 1), lambda i: (i, 0)),
                  pl.BlockSpec((D, D), lambda i: (0, 0))],
        out_specs=[pl.BlockSpec((NCH, R, CW), lambda i: (0, i, 0)),
                   pl.BlockSpec((R, D), lambda i: (i, 0))],
        out_shape=[jax.ShapeDtypeStruct((NCH, NP, CW), jnp.float32),
                   jax.ShapeDtypeStruct((N, D), jnp.float32)],
    )(agg, hp, bc, dinv, w)


def _postA_body(agg_ref, hp_ref, bc_ref, dinv_ref, w_ref, b_ref, v_ref, st_ref):
    x = _combine(agg_ref, hp_ref, bc_ref, dinv_ref)
    v = jnp.dot(x, w_ref[...], preferred_element_type=jnp.float32) + b_ref[...]
    v_ref[...] = v
    _acc_stats(st_ref, v, pl.program_id(0) == 0)


def _postA(agg, hp, bc, dinv, w, b):
    return pl.pallas_call(
        _postA_body,
        grid=(NR,),
        in_specs=[pl.BlockSpec((NCH, R, CW), lambda i: (0, i, 0)),
                  pl.BlockSpec((R, D), lambda i: (i, 0)),
                  pl.BlockSpec((1, D), lambda i: (0, 0)),
                  pl.BlockSpec((R, 1), lambda i: (i, 0)),
                  pl.BlockSpec((D, D), lambda i: (0, 0)),
                  pl.BlockSpec((1, D), lambda i: (0, 0))],
        out_specs=[pl.BlockSpec((R, D), lambda i: (i, 0)),
                   pl.BlockSpec((2, D), lambda i: (0, 0))],
        out_shape=[jax.ShapeDtypeStruct((N, D), jnp.float32),
                   jax.ShapeDtypeStruct((2, D), jnp.float32)],
    )(agg, hp, bc, dinv, w, b)


def _postB_body(u_ref, pst_ref, g_ref, be_ref, w_ref, b_ref, v_ref, st_ref):
    a, c = _affine_from_stats(pst_ref, g_ref, be_ref)
    x = jnp.maximum(u_ref[...] * a + c, 0.0)
    v = jnp.dot(x, w_ref[...], preferred_element_type=jnp.float32) + b_ref[...]
    v_ref[...] = v
    _acc_stats(st_ref, v, pl.program_id(0) == 0)


def _postB(u, pst, g, be, w, b):
    return pl.pallas_call(
        _postB_body,
        grid=(NR,),
        in_specs=[pl.BlockSpec((R, D), lambda i: (i, 0)),
                  pl.BlockSpec((2, D), lambda i: (0, 0)),
                  pl.BlockSpec((1, D), lambda i: (0, 0)),
                  pl.BlockSpec((1, D), lambda i: (0, 0)),
                  pl.BlockSpec((D, D), lambda i: (0, 0)),
                  pl.BlockSpec((1, D), lambda i: (0, 0))],
        out_specs=[pl.BlockSpec((R, D), lambda i: (i, 0)),
                   pl.BlockSpec((2, D), lambda i: (0, 0))],
        out_shape=[jax.ShapeDtypeStruct((N, D), jnp.float32),
                   jax.ShapeDtypeStruct((2, D), jnp.float32)],
    )(u, pst, g, be, w, b)


def _pool_body(v_ref, pst_ref, g_ref, be_ref, bt_ref, s_ref, c_ref):
    a, c0 = _affine_from_stats(pst_ref, g_ref, be_ref)
    xn = v_ref[...] * a + c0
    oh = (bt_ref[...] == lax.broadcasted_iota(jnp.int32, (1, NB), 1))
    oh = oh.astype(jnp.float32)

    @pl.when(pl.program_id(0) == 0)
    def _():
        s_ref[...] = jnp.zeros_like(s_ref)
        c_ref[...] = jnp.zeros_like(c_ref)

    dn = (((0,), (0,)), ((), ()))
    s_ref[...] += lax.dot_general(oh, xn, dn,
                                  preferred_element_type=jnp.float32)
    c_ref[...] += lax.dot_general(oh, jnp.ones_like(xn), dn,
                                  preferred_element_type=jnp.float32)


def _pool(v, pst, g, be, bt):
    return pl.pallas_call(
        _pool_body,
        grid=(NR,),
        in_specs=[pl.BlockSpec((R, D), lambda i: (i, 0)),
                  pl.BlockSpec((2, D), lambda i: (0, 0)),
                  pl.BlockSpec((1, D), lambda i: (0, 0)),
                  pl.BlockSpec((1, D), lambda i: (0, 0)),
                  pl.BlockSpec((R, 1), lambda i: (i, 0))],
        out_specs=[pl.BlockSpec((NB, D), lambda i: (0, 0)),
                   pl.BlockSpec((NB, D), lambda i: (0, 0))],
        out_shape=[jax.ShapeDtypeStruct((NB, D), jnp.float32),
                   jax.ShapeDtypeStruct((NB, D), jnp.float32)],
    )(v, pst, g, be, bt)


def _final_body(s1_ref, c1_ref, s2_ref, c2_ref, w1a_ref, w1b_ref, b1_ref,
                w2_ref, b2_ref, w3_ref, b3_ref, out_ref):
    m1 = s1_ref[...] / jnp.maximum(c1_ref[...], 1.0)
    m2 = s2_ref[...] / jnp.maximum(c2_ref[...], 1.0)
    g = jnp.dot(m1, w1a_ref[...], preferred_element_type=jnp.float32)
    g += jnp.dot(m2, w1b_ref[...], preferred_element_type=jnp.float32)
    g = jnp.maximum(g + b1_ref[...], 0.0)
    g = jnp.maximum(jnp.dot(g, w2_ref[...],
                            preferred_element_type=jnp.float32) + b2_ref[...], 0.0)
    out_ref[...] = jnp.dot(g, w3_ref[...],
                           preferred_element_type=jnp.float32) + b3_ref[...]


def _final(s1, c1, s2, c2, w1a, w1b, b1, w2, b2, w3p, b3p):
    return pl.pallas_call(
        _final_body,
        out_shape=jax.ShapeDtypeStruct((NB, D), jnp.float32),
    )(s1, c1, s2, c2, w1a, w1b, b1, w2, b2, w3p, b3p)


def _dinv_body(dg_ref, out_ref):
    out_ref[...] = lax.rsqrt(dg_ref[:, 0:1] + 1.0)


def _dinv(deg4):
    return pl.pallas_call(
        _dinv_body,
        grid=(NR,),
        in_specs=[pl.BlockSpec((R, DEGW), lambda i: (i, 0))],
        out_specs=pl.BlockSpec((R, 1), lambda i: (i, 0)),
        out_shape=jax.ShapeDtypeStruct((N, 1), jnp.float32),
    )(deg4)


def kernel(x_graph_1, x_graph_2, edge_index_g1, edge_index_g2, batch_g1,
           batch_g2, pre_W, pre_b, pre_gamma, pre_beta, conv_W, conv_b,
           post_W, post_b, post_gamma, post_beta, fin_W1, fin_b1, fin_W2,
           fin_b2, fin_W3, fin_b3):
    zrows = jnp.zeros((NP, DEGW), jnp.float32)
    orows = jnp.ones((B, DEGW), jnp.float32)
    pad = jnp.full((EPAD - E,), N, jnp.int32)

    pooled = []
    for j, (x, ei, bt) in enumerate(((x_graph_1, edge_index_g1, batch_g1),
                                     (x_graph_2, edge_index_g2, batch_g2))):
        srcr = jnp.concatenate([ei[0], pad])
        dstr = jnp.concatenate([ei[1], pad])

        deg4 = _sc_degree(dstr, zrows, orows)
        dinv = _dinv(deg4)

        u0, st0 = _mm_stats(x, pre_W[2, j], pre_b[2, j][None])

        y, h = _conv1(u0, st0, pre_gamma[2, j][None], pre_beta[2, j][None],
                      conv_W[0, j], dinv)
        agg = _sc_scatter(y, srcr, dstr)
        y, h = _conv23(agg, h, conv_b[0, j][None], dinv, conv_W[1, j])
        agg = _sc_scatter(y, srcr, dstr)
        y, h = _conv23(agg, h, conv_b[1, j][None], dinv, conv_W[2, j])
        agg = _sc_scatter(y, srcr, dstr)

        v, st = _postA(agg, h, conv_b[2, j][None], dinv,
                       post_W[0, j], post_b[0, j][None])
        v, st = _postB(v, st, post_gamma[0, j][None], post_beta[0, j][None],
                       post_W[1, j], post_b[1, j][None])
        v, st = _postB(v, st, post_gamma[1, j][None], post_beta[1, j][None],
                       post_W[2, j], post_b[2, j][None])
        s, c = _pool(v, st, post_gamma[2, j][None], post_beta[2, j][None],
                     bt.reshape(N, 1))
        pooled.append((s, c))

    w3p = jnp.pad(fin_W3, ((0, 0), (0, D - OUT)))
    b3p = jnp.pad(fin_b3, (0, D - OUT))[None]
    out = _final(pooled[0][0], pooled[0][1], pooled[1][0], pooled[1][1],
                 fin_W1[:D], fin_W1[D:], fin_b1[None], fin_W2, fin_b2[None],
                 w3p, b3p)
    return out[:, :OUT]


# speedup vs baseline: 1.0836x; 1.0836x over previous
import functools

import jax
import jax.numpy as jnp
from jax import lax
from jax.experimental import pallas as pl
from jax.experimental.pallas import tpu as pltpu
from jax.experimental.pallas import tpu_sc as plsc

N = 50000
E = 800000
D = 128
NB = 16
OUT = 7
EPS = 1e-5

R = 2000
NR = N // R
NCH = 4
CW = D // NCH
NP = N + 48
B = 400
EB = 51200
EPAD = 16 * EB
NBATCH = EB // B
TROWS = NP // 16
DEGW = 16

def _mesh():
    return plsc.VectorSubcoreMesh(core_axis_name="c", subcore_axis_name="s")


def _sc_degree(dstr, zrows, orows):
    @functools.partial(
        pl.kernel,
        mesh=_mesh(),
        compiler_params=pltpu.CompilerParams(use_tc_tiling_on_sc=False),
        out_type=jax.ShapeDtypeStruct((NP, DEGW), jnp.float32),
        scratch_types=[
            pltpu.VMEM((B,), jnp.int32),
            pltpu.VMEM((B, DEGW), jnp.float32),
            pltpu.VMEM_SHARED((NP, DEGW), jnp.float32),
        ],
    )
    def k(dst_hbm, z_hbm, one_hbm, out_hbm, di, ones_v, buf):
        cid = lax.axis_index("c")
        sid = lax.axis_index("s")

        @pl.when(cid == 0)
        def _():
            pltpu.sync_copy(z_hbm.at[pl.ds(sid * TROWS, TROWS)],
                            buf.at[pl.ds(sid * TROWS, TROWS)])
            pltpu.sync_copy(one_hbm, ones_v)
            plsc.subcore_barrier()

            def body(i, carry):
                e0 = sid * EB + i * B
                pltpu.sync_copy(dst_hbm.at[pl.ds(e0, B)], di)
                pltpu.sync_copy(ones_v, buf.at[di], add=True)
                return carry

            lax.fori_loop(0, NBATCH, body, 0)
            plsc.subcore_barrier()
            pltpu.sync_copy(buf.at[pl.ds(sid * TROWS, TROWS)],
                            out_hbm.at[pl.ds(sid * TROWS, TROWS)])

    return k(dstr, zrows, orows)


def _sc_scatter(y, srcr, dstr):
    @functools.partial(
        pl.kernel,
        mesh=_mesh(),
        compiler_params=pltpu.CompilerParams(use_tc_tiling_on_sc=False),
        out_type=jax.ShapeDtypeStruct((NCH, NP, CW), jnp.float32),
        scratch_types=[
            [pltpu.VMEM((B,), jnp.int32)] * 2,
            [pltpu.VMEM((B,), jnp.int32)] * 2,
            [pltpu.VMEM((B, CW), jnp.float32)] * 2,
            pltpu.VMEM_SHARED((NP, CW), jnp.float32),
            [pltpu.SemaphoreType.DMA] * 2,
            [pltpu.SemaphoreType.DMA] * 2,
            [pltpu.SemaphoreType.DMA] * 2,
        ],
    )
    def k(y_hbm, src_hbm, dst_hbm, out_hbm, si, di, rows, buf, isem, gsem,
          ssem):
        cid = lax.axis_index("c")
        sid = lax.axis_index("s")

        def issue_idx(i, b):
            e0 = sid * EB + i * B
            pltpu.async_copy(src_hbm.at[pl.ds(e0, B)], si[b], isem[b])
            pltpu.async_copy(dst_hbm.at[pl.ds(e0, B)], di[b], isem[b])

        def wait_idx(b):
            pltpu.make_async_copy(src_hbm.at[pl.ds(0, B)], si[b],
                                  isem[b]).wait()
            pltpu.make_async_copy(dst_hbm.at[pl.ds(0, B)], di[b],
                                  isem[b]).wait()

        def wait_scatter(b):
            pltpu.make_async_copy(rows[b], buf.at[di[b]], ssem[b]).wait()

        for kk in range(NCH // 2):
            ch = cid * (NCH // 2) + kk
            pltpu.sync_copy(y_hbm.at[ch].at[pl.ds(sid * TROWS, TROWS)],
                            buf.at[pl.ds(sid * TROWS, TROWS)])
            plsc.subcore_barrier()

            issue_idx(0, 0)

            def body(i2, carry):
                for b in range(2):
                    i = 2 * i2 + b
                    wait_idx(b)
                    pltpu.async_copy(y_hbm.at[ch].at[si[b]], rows[b], gsem[b])

                    @pl.when(i >= 1)
                    def _():
                        wait_scatter(1 - b)

                    @pl.when(i + 1 < NBATCH)
                    def _():
                        issue_idx(i + 1, 1 - b)

                    pltpu.make_async_copy(y_hbm.at[ch].at[si[b]], rows[b],
                                          gsem[b]).wait()
                    pltpu.async_copy(rows[b], buf.at[di[b]], ssem[b],
                                     add=True)
                return carry

            lax.fori_loop(0, NBATCH // 2, body, 0)
            wait_scatter(1)
            plsc.subcore_barrier()
            pltpu.sync_copy(buf.at[pl.ds(sid * TROWS, TROWS)],
                            out_hbm.at[ch].at[pl.ds(sid * TROWS, TROWS)])

    return k(y, srcr, dstr)


def _affine_from_stats(st_ref, g_ref, be_ref):
    m = st_ref[0:1, :] * (1.0 / N)
    var = st_ref[1:2, :] * (1.0 / N) - m * m
    a = g_ref[...] * lax.rsqrt(var + EPS)
    c = be_ref[...] - m * a
    return a, c


def _acc_stats(st_ref, o, first):
    @pl.when(first)
    def _():
        st_ref[...] = jnp.zeros_like(st_ref)
    st_ref[0:1, :] += jnp.sum(o, axis=0, keepdims=True)
    st_ref[1:2, :] += jnp.sum(o * o, axis=0, keepdims=True)


def _mm_stats_body(x_ref, w_ref, b_ref, out_ref, st_ref):
    o = jnp.dot(x_ref[...], w_ref[...],
                preferred_element_type=jnp.float32) + b_ref[...]
    out_ref[...] = o
    _acc_stats(st_ref, o, pl.program_id(0) == 0)


def _mm_stats(x, w, b):
    return pl.pallas_call(
        _mm_stats_body,
        grid=(NR,),
        in_specs=[pl.BlockSpec((R, D), lambda i: (i, 0)),
                  pl.BlockSpec((D, D), lambda i: (0, 0)),
                  pl.BlockSpec((1, D), lambda i: (0, 0))],
        out_specs=[pl.BlockSpec((R, D), lambda i: (i, 0)),
                   pl.BlockSpec((2, D), lambda i: (0, 0))],
        out_shape=[jax.ShapeDtypeStruct((N, D), jnp.float32),
                   jax.ShapeDtypeStruct((2, D), jnp.float32)],
    )(x, w, b)


def _store_chunked(y_ref, yfull):
    for c in range(NCH):
        y_ref[c, :, :] = yfull[:, c * CW:(c + 1) * CW]


def _conv1_body(u_ref, st_ref, g_ref, be_ref, w_ref, dinv_ref, y_ref, h_ref):
    a, c = _affine_from_stats(st_ref, g_ref, be_ref)
    h = jnp.maximum(u_ref[...] * a + c, 0.0)
    h_ref[...] = h
    _store_chunked(y_ref, jnp.dot(h, w_ref[...],
                                  preferred_element_type=jnp.float32)
                   * dinv_ref[...])


def _conv1(u, st, g, be, w, dinv):
    return pl.pallas_call(
        _conv1_body,
        grid=(NR,),
        in_specs=[pl.BlockSpec((R, D), lambda i: (i, 0)),
                  pl.BlockSpec((2, D), lambda i: (0, 0)),
                  pl.BlockSpec((1, D), lambda i: (0, 0)),
                  pl.BlockSpec((1, D), lambda i: (0, 0)),
                  pl.BlockSpec((D, D), lambda i: (0, 0)),
                  pl.BlockSpec((R, 1), lambda i: (i, 0))],
        out_specs=[pl.BlockSpec((NCH, R, CW), lambda i: (0, i, 0)),
                   pl.BlockSpec((R, D), lambda i: (i, 0))],
        out_shape=[jax.ShapeDtypeStruct((NCH, NP, CW), jnp.float32),
                   jax.ShapeDtypeStruct((N, D), jnp.float32)],
    )(u, st, g, be, w, dinv)


def _combine(agg_ref, hp_ref, bc_ref, dinv_ref):
    agg = jnp.concatenate([agg_ref[kk] for kk in range(NCH)], axis=1)
    return jnp.maximum(agg * dinv_ref[...] + bc_ref[...] + hp_ref[...], 0.0)


def _conv23_body(agg_ref, hp_ref, bc_ref, dinv_ref, w_ref, y_ref, h_ref):
    x = _combine(agg_ref, hp_ref, bc_ref, dinv_ref)
    h_ref[...] = x
    _store_chunked(y_ref, jnp.dot(x, w_ref[...],
                                  preferred_element_type=jnp.float32)
                   * dinv_ref[...])


def _conv23(agg, hp, bc, dinv, w):
    return pl.pallas_call(
        _conv23_body,
        grid=(NR,),
        in_specs=[pl.BlockSpec((NCH, R, CW), lambda i: (0, i, 0)),
                  pl.BlockSpec((R, D), lambda i: (i, 0)),
                  pl.BlockSpec((1, D), lambda i: (0, 0)),
                  pl.BlockSpec((R, 1), lambda i: (i, 0)),
                  pl.BlockSpec((D, D), lambda i: (0, 0))],
        out_specs=[pl.BlockSpec((NCH, R, CW), lambda i: (0, i, 0)),
                   pl.BlockSpec((R, D), lambda i: (i, 0))],
        out_shape=[jax.ShapeDtypeStruct((NCH, NP, CW), jnp.float32),
                   jax.ShapeDtypeStruct((N, D), jnp.float32)],
    )(agg, hp, bc, dinv, w)


def _postA_body(agg_ref, hp_ref, bc_ref, dinv_ref, w_ref, b_ref, v_ref, st_ref):
    x = _combine(agg_ref, hp_ref, bc_ref, dinv_ref)
    v = jnp.dot(x, w_ref[...], preferred_element_type=jnp.float32) + b_ref[...]
    v_ref[...] = v
    _acc_stats(st_ref, v, pl.program_id(0) == 0)


def _postA(agg, hp, bc, dinv, w, b):
    return pl.pallas_call(
        _postA_body,
        grid=(NR,),
        in_specs=[pl.BlockSpec((NCH, R, CW), lambda i: (0, i, 0)),
                  pl.BlockSpec((R, D), lambda i: (i, 0)),
                  pl.BlockSpec((1, D), lambda i: (0, 0)),
                  pl.BlockSpec((R, 1), lambda i: (i, 0)),
                  pl.BlockSpec((D, D), lambda i: (0, 0)),
                  pl.BlockSpec((1, D), lambda i: (0, 0))],
        out_specs=[pl.BlockSpec((R, D), lambda i: (i, 0)),
                   pl.BlockSpec((2, D), lambda i: (0, 0))],
        out_shape=[jax.ShapeDtypeStruct((N, D), jnp.float32),
                   jax.ShapeDtypeStruct((2, D), jnp.float32)],
    )(agg, hp, bc, dinv, w, b)


def _postB_body(u_ref, pst_ref, g_ref, be_ref, w_ref, b_ref, v_ref, st_ref):
    a, c = _affine_from_stats(pst_ref, g_ref, be_ref)
    x = jnp.maximum(u_ref[...] * a + c, 0.0)
    v = jnp.dot(x, w_ref[...], preferred_element_type=jnp.float32) + b_ref[...]
    v_ref[...] = v
    _acc_stats(st_ref, v, pl.program_id(0) == 0)


def _postB(u, pst, g, be, w, b):
    return pl.pallas_call(
        _postB_body,
        grid=(NR,),
        in_specs=[pl.BlockSpec((R, D), lambda i: (i, 0)),
                  pl.BlockSpec((2, D), lambda i: (0, 0)),
                  pl.BlockSpec((1, D), lambda i: (0, 0)),
                  pl.BlockSpec((1, D), lambda i: (0, 0)),
                  pl.BlockSpec((D, D), lambda i: (0, 0)),
                  pl.BlockSpec((1, D), lambda i: (0, 0))],
        out_specs=[pl.BlockSpec((R, D), lambda i: (i, 0)),
                   pl.BlockSpec((2, D), lambda i: (0, 0))],
        out_shape=[jax.ShapeDtypeStruct((N, D), jnp.float32),
                   jax.ShapeDtypeStruct((2, D), jnp.float32)],
    )(u, pst, g, be, w, b)


def _pool_body(v_ref, pst_ref, g_ref, be_ref, bt_ref, s_ref, c_ref):
    a, c0 = _affine_from_stats(pst_ref, g_ref, be_ref)
    xn = v_ref[...] * a + c0
    oh = (bt_ref[...] == lax.broadcasted_iota(jnp.int32, (1, NB), 1))
    oh = oh.astype(jnp.float32)

    @pl.when(pl.program_id(0) == 0)
    def _():
        s_ref[...] = jnp.zeros_like(s_ref)
        c_ref[...] = jnp.zeros_like(c_ref)

    dn = (((0,), (0,)), ((), ()))
    s_ref[...] += lax.dot_general(oh, xn, dn,
                                  preferred_element_type=jnp.float32)
    c_ref[...] += lax.dot_general(oh, jnp.ones_like(xn), dn,
                                  preferred_element_type=jnp.float32)


def _pool(v, pst, g, be, bt):
    return pl.pallas_call(
        _pool_body,
        grid=(NR,),
        in_specs=[pl.BlockSpec((R, D), lambda i: (i, 0)),
                  pl.BlockSpec((2, D), lambda i: (0, 0)),
                  pl.BlockSpec((1, D), lambda i: (0, 0)),
                  pl.BlockSpec((1, D), lambda i: (0, 0)),
                  pl.BlockSpec((R, 1), lambda i: (i, 0))],
        out_specs=[pl.BlockSpec((NB, D), lambda i: (0, 0)),
                   pl.BlockSpec((NB, D), lambda i: (0, 0))],
        out_shape=[jax.ShapeDtypeStruct((NB, D), jnp.float32),
                   jax.ShapeDtypeStruct((NB, D), jnp.float32)],
    )(v, pst, g, be, bt)


def _final_body(s1_ref, c1_ref, s2_ref, c2_ref, w1a_ref, w1b_ref, b1_ref,
                w2_ref, b2_ref, w3_ref, b3_ref, out_ref):
    m1 = s1_ref[...] / jnp.maximum(c1_ref[...], 1.0)
    m2 = s2_ref[...] / jnp.maximum(c2_ref[...], 1.0)
    g = jnp.dot(m1, w1a_ref[...], preferred_element_type=jnp.float32)
    g += jnp.dot(m2, w1b_ref[...], preferred_element_type=jnp.float32)
    g = jnp.maximum(g + b1_ref[...], 0.0)
    g = jnp.maximum(jnp.dot(g, w2_ref[...],
                            preferred_element_type=jnp.float32) + b2_ref[...], 0.0)
    out_ref[...] = jnp.dot(g, w3_ref[...],
                           preferred_element_type=jnp.float32) + b3_ref[...]


def _final(s1, c1, s2, c2, w1a, w1b, b1, w2, b2, w3p, b3p):
    return pl.pallas_call(
        _final_body,
        out_shape=jax.ShapeDtypeStruct((NB, D), jnp.float32),
    )(s1, c1, s2, c2, w1a, w1b, b1, w2, b2, w3p, b3p)


def _dinv_body(dg_ref, out_ref):
    out_ref[...] = lax.rsqrt(dg_ref[:, 0:1] + 1.0)


def _dinv(deg4):
    return pl.pallas_call(
        _dinv_body,
        grid=(NR,),
        in_specs=[pl.BlockSpec((R, DEGW), lambda i: (i, 0))],
        out_specs=pl.BlockSpec((R, 1), lambda i: (i, 0)),
        out_shape=jax.ShapeDtypeStruct((N, 1), jnp.float32),
    )(deg4)


def kernel(x_graph_1, x_graph_2, edge_index_g1, edge_index_g2, batch_g1,
           batch_g2, pre_W, pre_b, pre_gamma, pre_beta, conv_W, conv_b,
           post_W, post_b, post_gamma, post_beta, fin_W1, fin_b1, fin_W2,
           fin_b2, fin_W3, fin_b3):
    zrows = jnp.zeros((NP, DEGW), jnp.float32)
    orows = jnp.ones((B, DEGW), jnp.float32)
    pad = jnp.full((EPAD - E,), N, jnp.int32)

    pooled = []
    for j, (x, ei, bt) in enumerate(((x_graph_1, edge_index_g1, batch_g1),
                                     (x_graph_2, edge_index_g2, batch_g2))):
        srcr = jnp.concatenate([ei[0], pad])
        dstr = jnp.concatenate([ei[1], pad])

        deg4 = _sc_degree(dstr, zrows, orows)
        dinv = _dinv(deg4)

        u0, st0 = _mm_stats(x, pre_W[2, j], pre_b[2, j][None])

        y, h = _conv1(u0, st0, pre_gamma[2, j][None], pre_beta[2, j][None],
                      conv_W[0, j], dinv)
        agg = _sc_scatter(y, srcr, dstr)
        y, h = _conv23(agg, h, conv_b[0, j][None], dinv, conv_W[1, j])
        agg = _sc_scatter(y, srcr, dstr)
        y, h = _conv23(agg, h, conv_b[1, j][None], dinv, conv_W[2, j])
        agg = _sc_scatter(y, srcr, dstr)

        v, st = _postA(agg, h, conv_b[2, j][None], dinv,
                       post_W[0, j], post_b[0, j][None])
        v, st = _postB(v, st, post_gamma[0, j][None], post_beta[0, j][None],
                       post_W[1, j], post_b[1, j][None])
        v, st = _postB(v, st, post_gamma[1, j][None], post_beta[1, j][None],
                       post_W[2, j], post_b[2, j][None])
        s, c = _pool(v, st, post_gamma[2, j][None], post_beta[2, j][None],
                     bt.reshape(N, 1))
        pooled.append((s, c))

    w3p = jnp.pad(fin_W3, ((0, 0), (0, D - OUT)))
    b3p = jnp.pad(fin_b3, (0, D - OUT))[None]
    out = _final(pooled[0][0], pooled[0][1], pooled[1][0], pooled[1][1],
                 fin_W1[:D], fin_W1[D:], fin_b1[None], fin_W2, fin_b2[None],
                 w3p, b3p)
    return out[:, :OUT]


# speedup vs baseline: 1.3053x; 1.2046x over previous
import functools

import jax
import jax.numpy as jnp
from jax import lax
from jax.experimental import pallas as pl
from jax.experimental.pallas import tpu as pltpu
from jax.experimental.pallas import tpu_sc as plsc

N = 50000
E = 800000
D = 128
NB = 16
OUT = 7
EPS = 1e-5

R = 2000
NR = N // R
NCH = 4
CW = D // NCH
NP = N + 48
B = 256
EB = 50688
EPAD = 16 * EB
NBATCH = EB // B
TROWS = NP // 16
DEGW = 16

def _mesh():
    return plsc.VectorSubcoreMesh(core_axis_name="c", subcore_axis_name="s")


def _sc_degree(dstr, zrows, orows):
    @functools.partial(
        pl.kernel,
        mesh=_mesh(),
        compiler_params=pltpu.CompilerParams(use_tc_tiling_on_sc=False),
        out_type=jax.ShapeDtypeStruct((NP, DEGW), jnp.float32),
        scratch_types=[
            pltpu.VMEM((B,), jnp.int32),
            pltpu.VMEM((B, DEGW), jnp.float32),
            pltpu.VMEM_SHARED((NP, DEGW), jnp.float32),
        ],
    )
    def k(dst_hbm, z_hbm, one_hbm, out_hbm, di, ones_v, buf):
        cid = lax.axis_index("c")
        sid = lax.axis_index("s")

        @pl.when(cid == 0)
        def _():
            pltpu.sync_copy(z_hbm.at[pl.ds(sid * TROWS, TROWS)],
                            buf.at[pl.ds(sid * TROWS, TROWS)])
            pltpu.sync_copy(one_hbm, ones_v)
            plsc.subcore_barrier()

            def body(i, carry):
                e0 = sid * EB + i * B
                pltpu.sync_copy(dst_hbm.at[pl.ds(e0, B)], di)
                pltpu.sync_copy(ones_v, buf.at[di], add=True)
                return carry

            lax.fori_loop(0, NBATCH, body, 0)
            plsc.subcore_barrier()
            pltpu.sync_copy(buf.at[pl.ds(sid * TROWS, TROWS)],
                            out_hbm.at[pl.ds(sid * TROWS, TROWS)])

    return k(dstr, zrows, orows)


def _sc_scatter(y, srcr, dstr):
    @functools.partial(
        pl.kernel,
        mesh=_mesh(),
        compiler_params=pltpu.CompilerParams(use_tc_tiling_on_sc=False),
        out_type=jax.ShapeDtypeStruct((NCH, NP, CW), jnp.float32),
        scratch_types=[
            [pltpu.VMEM((B,), jnp.int32)] * 3,
            [pltpu.VMEM((B,), jnp.int32)] * 3,
            [pltpu.VMEM((B, CW), jnp.float32)] * 3,
            pltpu.VMEM_SHARED((NP, CW), jnp.float32),
            [pltpu.SemaphoreType.DMA] * 3,
            [pltpu.SemaphoreType.DMA] * 3,
            [pltpu.SemaphoreType.DMA] * 3,
        ],
    )
    def k(y_hbm, src_hbm, dst_hbm, out_hbm, si, di, rows, buf, isem, gsem,
          ssem):
        cid = lax.axis_index("c")
        sid = lax.axis_index("s")

        def issue_idx(i, b):
            e0 = sid * EB + i * B
            pltpu.async_copy(src_hbm.at[pl.ds(e0, B)], si[b], isem[b])
            pltpu.async_copy(dst_hbm.at[pl.ds(e0, B)], di[b], isem[b])

        def wait_idx(b):
            pltpu.make_async_copy(src_hbm.at[pl.ds(0, B)], si[b],
                                  isem[b]).wait()
            pltpu.make_async_copy(dst_hbm.at[pl.ds(0, B)], di[b],
                                  isem[b]).wait()

        def wait_scatter(b):
            pltpu.make_async_copy(rows[b], buf.at[di[b]], ssem[b]).wait()

        for kk in range(NCH // 2):
            ch = cid * (NCH // 2) + kk
            pltpu.sync_copy(y_hbm.at[ch].at[pl.ds(sid * TROWS, TROWS)],
                            buf.at[pl.ds(sid * TROWS, TROWS)])
            plsc.subcore_barrier()

            issue_idx(0, 0)

            def body(i3, carry):
                for b in range(3):
                    i = 3 * i3 + b
                    wait_idx(b)
                    pltpu.async_copy(y_hbm.at[ch].at[si[b]], rows[b], gsem[b])

                    @pl.when(i >= 2)
                    def _():
                        wait_scatter((b + 1) % 3)

                    @pl.when(i + 1 < NBATCH)
                    def _():
                        issue_idx(i + 1, (b + 1) % 3)

                    pltpu.make_async_copy(y_hbm.at[ch].at[si[b]], rows[b],
                                          gsem[b]).wait()
                    pltpu.async_copy(rows[b], buf.at[di[b]], ssem[b],
                                     add=True)
                return carry

            lax.fori_loop(0, NBATCH // 3, body, 0)
            wait_scatter((NBATCH - 2) % 3)
            wait_scatter((NBATCH - 1) % 3)
            plsc.subcore_barrier()
            pltpu.sync_copy(buf.at[pl.ds(sid * TROWS, TROWS)],
                            out_hbm.at[ch].at[pl.ds(sid * TROWS, TROWS)])

    return k(y, srcr, dstr)


def _affine_from_stats(st_ref, g_ref, be_ref):
    m = st_ref[0:1, :] * (1.0 / N)
    var = st_ref[1:2, :] * (1.0 / N) - m * m
    a = g_ref[...] * lax.rsqrt(var + EPS)
    c = be_ref[...] - m * a
    return a, c


def _acc_stats(st_ref, o, first):
    @pl.when(first)
    def _():
        st_ref[...] = jnp.zeros_like(st_ref)
    st_ref[0:1, :] += jnp.sum(o, axis=0, keepdims=True)
    st_ref[1:2, :] += jnp.sum(o * o, axis=0, keepdims=True)


def _mm_stats_body(x_ref, w_ref, b_ref, out_ref, st_ref):
    o = jnp.dot(x_ref[...], w_ref[...],
                preferred_element_type=jnp.float32) + b_ref[...]
    out_ref[...] = o
    _acc_stats(st_ref, o, pl.program_id(0) == 0)


def _mm_stats(x, w, b):
    return pl.pallas_call(
        _mm_stats_body,
        grid=(NR,),
        in_specs=[pl.BlockSpec((R, D), lambda i: (i, 0)),
                  pl.BlockSpec((D, D), lambda i: (0, 0)),
                  pl.BlockSpec((1, D), lambda i: (0, 0))],
        out_specs=[pl.BlockSpec((R, D), lambda i: (i, 0)),
                   pl.BlockSpec((2, D), lambda i: (0, 0))],
        out_shape=[jax.ShapeDtypeStruct((N, D), jnp.float32),
                   jax.ShapeDtypeStruct((2, D), jnp.float32)],
    )(x, w, b)


def _store_chunked(y_ref, yfull):
    for c in range(NCH):
        y_ref[c, :, :] = yfull[:, c * CW:(c + 1) * CW]


def _conv1_body(u_ref, st_ref, g_ref, be_ref, w_ref, dinv_ref, y_ref, h_ref):
    a, c = _affine_from_stats(st_ref, g_ref, be_ref)
    h = jnp.maximum(u_ref[...] * a + c, 0.0)
    h_ref[...] = h
    _store_chunked(y_ref, jnp.dot(h, w_ref[...],
                                  preferred_element_type=jnp.float32)
                   * dinv_ref[...])


def _conv1(u, st, g, be, w, dinv):
    return pl.pallas_call(
        _conv1_body,
        grid=(NR,),
        in_specs=[pl.BlockSpec((R, D), lambda i: (i, 0)),
                  pl.BlockSpec((2, D), lambda i: (0, 0)),
                  pl.BlockSpec((1, D), lambda i: (0, 0)),
                  pl.BlockSpec((1, D), lambda i: (0, 0)),
                  pl.BlockSpec((D, D), lambda i: (0, 0)),
                  pl.BlockSpec((R, 1), lambda i: (i, 0))],
        out_specs=[pl.BlockSpec((NCH, R, CW), lambda i: (0, i, 0)),
                   pl.BlockSpec((R, D), lambda i: (i, 0))],
        out_shape=[jax.ShapeDtypeStruct((NCH, NP, CW), jnp.float32),
                   jax.ShapeDtypeStruct((N, D), jnp.float32)],
    )(u, st, g, be, w, dinv)


def _combine(agg_ref, hp_ref, bc_ref, dinv_ref):
    agg = jnp.concatenate([agg_ref[kk] for kk in range(NCH)], axis=1)
    return jnp.maximum(agg * dinv_ref[...] + bc_ref[...] + hp_ref[...], 0.0)


def _conv23_body(agg_ref, hp_ref, bc_ref, dinv_ref, w_ref, y_ref, h_ref):
    x = _combine(agg_ref, hp_ref, bc_ref, dinv_ref)
    h_ref[...] = x
    _store_chunked(y_ref, jnp.dot(x, w_ref[...],
                                  preferred_element_type=jnp.float32)
                   * dinv_ref[...])


def _conv23(agg, hp, bc, dinv, w):
    return pl.pallas_call(
        _conv23_body,
        grid=(NR,),
        in_specs=[pl.BlockSpec((NCH, R, CW), lambda i: (0, i, 0)),
                  pl.BlockSpec((R, D), lambda i: (i, 0)),
                  pl.BlockSpec((1, D), lambda i: (0, 0)),
                  pl.BlockSpec((R, 1), lambda i: (i, 0)),
                  pl.BlockSpec((D, D), lambda i: (0, 0))],
        out_specs=[pl.BlockSpec((NCH, R, CW), lambda i: (0, i, 0)),
                   pl.BlockSpec((R, D), lambda i: (i, 0))],
        out_shape=[jax.ShapeDtypeStruct((NCH, NP, CW), jnp.float32),
                   jax.ShapeDtypeStruct((N, D), jnp.float32)],
    )(agg, hp, bc, dinv, w)


def _postA_body(agg_ref, hp_ref, bc_ref, dinv_ref, w_ref, b_ref, v_ref, st_ref):
    x = _combine(agg_ref, hp_ref, bc_ref, dinv_ref)
    v = jnp.dot(x, w_ref[...], preferred_element_type=jnp.float32) + b_ref[...]
    v_ref[...] = v
    _acc_stats(st_ref, v, pl.program_id(0) == 0)


def _postA(agg, hp, bc, dinv, w, b):
    return pl.pallas_call(
        _postA_body,
        grid=(NR,),
        in_specs=[pl.BlockSpec((NCH, R, CW), lambda i: (0, i, 0)),
                  pl.BlockSpec((R, D), lambda i: (i, 0)),
                  pl.BlockSpec((1, D), lambda i: (0, 0)),
                  pl.BlockSpec((R, 1), lambda i: (i, 0)),
                  pl.BlockSpec((D, D), lambda i: (0, 0)),
                  pl.BlockSpec((1, D), lambda i: (0, 0))],
        out_specs=[pl.BlockSpec((R, D), lambda i: (i, 0)),
                   pl.BlockSpec((2, D), lambda i: (0, 0))],
        out_shape=[jax.ShapeDtypeStruct((N, D), jnp.float32),
                   jax.ShapeDtypeStruct((2, D), jnp.float32)],
    )(agg, hp, bc, dinv, w, b)


def _postB_body(u_ref, pst_ref, g_ref, be_ref, w_ref, b_ref, v_ref, st_ref):
    a, c = _affine_from_stats(pst_ref, g_ref, be_ref)
    x = jnp.maximum(u_ref[...] * a + c, 0.0)
    v = jnp.dot(x, w_ref[...], preferred_element_type=jnp.float32) + b_ref[...]
    v_ref[...] = v
    _acc_stats(st_ref, v, pl.program_id(0) == 0)


def _postB(u, pst, g, be, w, b):
    return pl.pallas_call(
        _postB_body,
        grid=(NR,),
        in_specs=[pl.BlockSpec((R, D), lambda i: (i, 0)),
                  pl.BlockSpec((2, D), lambda i: (0, 0)),
                  pl.BlockSpec((1, D), lambda i: (0, 0)),
                  pl.BlockSpec((1, D), lambda i: (0, 0)),
                  pl.BlockSpec((D, D), lambda i: (0, 0)),
                  pl.BlockSpec((1, D), lambda i: (0, 0))],
        out_specs=[pl.BlockSpec((R, D), lambda i: (i, 0)),
                   pl.BlockSpec((2, D), lambda i: (0, 0))],
        out_shape=[jax.ShapeDtypeStruct((N, D), jnp.float32),
                   jax.ShapeDtypeStruct((2, D), jnp.float32)],
    )(u, pst, g, be, w, b)


def _pool_body(v_ref, pst_ref, g_ref, be_ref, bt_ref, s_ref, c_ref):
    a, c0 = _affine_from_stats(pst_ref, g_ref, be_ref)
    xn = v_ref[...] * a + c0
    oh = (bt_ref[...] == lax.broadcasted_iota(jnp.int32, (1, NB), 1))
    oh = oh.astype(jnp.float32)

    @pl.when(pl.program_id(0) == 0)
    def _():
        s_ref[...] = jnp.zeros_like(s_ref)
        c_ref[...] = jnp.zeros_like(c_ref)

    dn = (((0,), (0,)), ((), ()))
    s_ref[...] += lax.dot_general(oh, xn, dn,
                                  preferred_element_type=jnp.float32)
    c_ref[...] += lax.dot_general(oh, jnp.ones_like(xn), dn,
                                  preferred_element_type=jnp.float32)


def _pool(v, pst, g, be, bt):
    return pl.pallas_call(
        _pool_body,
        grid=(NR,),
        in_specs=[pl.BlockSpec((R, D), lambda i: (i, 0)),
                  pl.BlockSpec((2, D), lambda i: (0, 0)),
                  pl.BlockSpec((1, D), lambda i: (0, 0)),
                  pl.BlockSpec((1, D), lambda i: (0, 0)),
                  pl.BlockSpec((R, 1), lambda i: (i, 0))],
        out_specs=[pl.BlockSpec((NB, D), lambda i: (0, 0)),
                   pl.BlockSpec((NB, D), lambda i: (0, 0))],
        out_shape=[jax.ShapeDtypeStruct((NB, D), jnp.float32),
                   jax.ShapeDtypeStruct((NB, D), jnp.float32)],
    )(v, pst, g, be, bt)


def _final_body(s1_ref, c1_ref, s2_ref, c2_ref, w1a_ref, w1b_ref, b1_ref,
                w2_ref, b2_ref, w3_ref, b3_ref, out_ref):
    m1 = s1_ref[...] / jnp.maximum(c1_ref[...], 1.0)
    m2 = s2_ref[...] / jnp.maximum(c2_ref[...], 1.0)
    g = jnp.dot(m1, w1a_ref[...], preferred_element_type=jnp.float32)
    g += jnp.dot(m2, w1b_ref[...], preferred_element_type=jnp.float32)
    g = jnp.maximum(g + b1_ref[...], 0.0)
    g = jnp.maximum(jnp.dot(g, w2_ref[...],
                            preferred_element_type=jnp.float32) + b2_ref[...], 0.0)
    out_ref[...] = jnp.dot(g, w3_ref[...],
                           preferred_element_type=jnp.float32) + b3_ref[...]


def _final(s1, c1, s2, c2, w1a, w1b, b1, w2, b2, w3p, b3p):
    return pl.pallas_call(
        _final_body,
        out_shape=jax.ShapeDtypeStruct((NB, D), jnp.float32),
    )(s1, c1, s2, c2, w1a, w1b, b1, w2, b2, w3p, b3p)


def _dinv_body(dg_ref, out_ref):
    out_ref[...] = lax.rsqrt(dg_ref[:, 0:1] + 1.0)


def _dinv(deg4):
    return pl.pallas_call(
        _dinv_body,
        grid=(NR,),
        in_specs=[pl.BlockSpec((R, DEGW), lambda i: (i, 0))],
        out_specs=pl.BlockSpec((R, 1), lambda i: (i, 0)),
        out_shape=jax.ShapeDtypeStruct((N, 1), jnp.float32),
    )(deg4)


def kernel(x_graph_1, x_graph_2, edge_index_g1, edge_index_g2, batch_g1,
           batch_g2, pre_W, pre_b, pre_gamma, pre_beta, conv_W, conv_b,
           post_W, post_b, post_gamma, post_beta, fin_W1, fin_b1, fin_W2,
           fin_b2, fin_W3, fin_b3):
    zrows = jnp.zeros((NP, DEGW), jnp.float32)
    orows = jnp.ones((B, DEGW), jnp.float32)
    pad = jnp.full((EPAD - E,), N, jnp.int32)

    pooled = []
    for j, (x, ei, bt) in enumerate(((x_graph_1, edge_index_g1, batch_g1),
                                     (x_graph_2, edge_index_g2, batch_g2))):
        srcr = jnp.concatenate([ei[0], pad])
        dstr = jnp.concatenate([ei[1], pad])

        deg4 = _sc_degree(dstr, zrows, orows)
        dinv = _dinv(deg4)

        u0, st0 = _mm_stats(x, pre_W[2, j], pre_b[2, j][None])

        y, h = _conv1(u0, st0, pre_gamma[2, j][None], pre_beta[2, j][None],
                      conv_W[0, j], dinv)
        agg = _sc_scatter(y, srcr, dstr)
        y, h = _conv23(agg, h, conv_b[0, j][None], dinv, conv_W[1, j])
        agg = _sc_scatter(y, srcr, dstr)
        y, h = _conv23(agg, h, conv_b[1, j][None], dinv, conv_W[2, j])
        agg = _sc_scatter(y, srcr, dstr)

        v, st = _postA(agg, h, conv_b[2, j][None], dinv,
                       post_W[0, j], post_b[0, j][None])
        v, st = _postB(v, st, post_gamma[0, j][None], post_beta[0, j][None],
                       post_W[1, j], post_b[1, j][None])
        v, st = _postB(v, st, post_gamma[1, j][None], post_beta[1, j][None],
                       post_W[2, j], post_b[2, j][None])
        s, c = _pool(v, st, post_gamma[2, j][None], post_beta[2, j][None],
                     bt.reshape(N, 1))
        pooled.append((s, c))

    w3p = jnp.pad(fin_W3, ((0, 0), (0, D - OUT)))
    b3p = jnp.pad(fin_b3, (0, D - OUT))[None]
    out = _final(pooled[0][0], pooled[0][1], pooled[1][0], pooled[1][1],
                 fin_W1[:D], fin_W1[D:], fin_b1[None], fin_W2, fin_b2[None],
                 w3p, b3p)
    return out[:, :OUT]


# speedup vs baseline: 1.4059x; 1.0771x over previous
import functools

import jax
import jax.numpy as jnp
from jax import lax
from jax.experimental import pallas as pl
from jax.experimental.pallas import tpu as pltpu
from jax.experimental.pallas import tpu_sc as plsc

N = 50000
E = 800000
D = 128
NB = 16
OUT = 7
EPS = 1e-5

R = 2000
NR = N // R
NCH = 4
CW = D // NCH
NP = N + 48
B = 256
EB = 50688
EPAD = 16 * EB
NBATCH = EB // B
TROWS = NP // 16
DEGW = 16

def _mesh():
    return plsc.VectorSubcoreMesh(core_axis_name="c", subcore_axis_name="s")


def _sc_degree(dstr, zrows, orows):
    @functools.partial(
        pl.kernel,
        mesh=_mesh(),
        compiler_params=pltpu.CompilerParams(use_tc_tiling_on_sc=False),
        out_type=jax.ShapeDtypeStruct((NP, DEGW), jnp.float32),
        scratch_types=[
            pltpu.VMEM((B,), jnp.int32),
            pltpu.VMEM((B, DEGW), jnp.float32),
            pltpu.VMEM_SHARED((NP, DEGW), jnp.float32),
        ],
    )
    def k(dst_hbm, z_hbm, one_hbm, out_hbm, di, ones_v, buf):
        cid = lax.axis_index("c")
        sid = lax.axis_index("s")

        @pl.when(cid == 0)
        def _():
            pltpu.sync_copy(z_hbm.at[pl.ds(sid * TROWS, TROWS)],
                            buf.at[pl.ds(sid * TROWS, TROWS)])
            pltpu.sync_copy(one_hbm, ones_v)
            plsc.subcore_barrier()

            def body(i, carry):
                e0 = sid * EB + i * B
                pltpu.sync_copy(dst_hbm.at[pl.ds(e0, B)], di)
                pltpu.sync_copy(ones_v, buf.at[di], add=True)
                return carry

            lax.fori_loop(0, NBATCH, body, 0)
            plsc.subcore_barrier()
            pltpu.sync_copy(buf.at[pl.ds(sid * TROWS, TROWS)],
                            out_hbm.at[pl.ds(sid * TROWS, TROWS)])

    return k(dstr, zrows, orows)


def _sc_scatter(y, srcr, dstr):
    @functools.partial(
        pl.kernel,
        mesh=_mesh(),
        compiler_params=pltpu.CompilerParams(use_tc_tiling_on_sc=False),
        out_type=jax.ShapeDtypeStruct((NCH, NP, CW), jnp.float32),
        scratch_types=[
            [pltpu.VMEM((B,), jnp.int32)] * 3,
            [pltpu.VMEM((B,), jnp.int32)] * 3,
            [pltpu.VMEM((B, CW), jnp.float32)] * 3,
            pltpu.VMEM_SHARED((NP, CW), jnp.float32),
            [pltpu.SemaphoreType.DMA] * 3,
            [pltpu.SemaphoreType.DMA] * 3,
            [pltpu.SemaphoreType.DMA] * 3,
        ],
    )
    def k(y_hbm, src_hbm, dst_hbm, out_hbm, si, di, rows, buf, isem, gsem,
          ssem):
        cid = lax.axis_index("c")
        sid = lax.axis_index("s")

        def issue_idx(i, b):
            e0 = sid * EB + i * B
            pltpu.async_copy(src_hbm.at[pl.ds(e0, B)], si[b], isem[b])
            pltpu.async_copy(dst_hbm.at[pl.ds(e0, B)], di[b], isem[b])

        def wait_idx(b):
            pltpu.make_async_copy(src_hbm.at[pl.ds(0, B)], si[b],
                                  isem[b]).wait()
            pltpu.make_async_copy(dst_hbm.at[pl.ds(0, B)], di[b],
                                  isem[b]).wait()

        def wait_scatter(b):
            pltpu.make_async_copy(rows[b], buf.at[di[b]], ssem[b]).wait()

        def retire_gather(b, ch):
            pltpu.make_async_copy(y_hbm.at[ch].at[si[b]], rows[b],
                                  gsem[b]).wait()
            pltpu.async_copy(rows[b], buf.at[di[b]], ssem[b], add=True)

        for kk in range(NCH // 2):
            ch = cid * (NCH // 2) + kk
            pltpu.sync_copy(y_hbm.at[ch].at[pl.ds(sid * TROWS, TROWS)],
                            buf.at[pl.ds(sid * TROWS, TROWS)])
            plsc.subcore_barrier()

            issue_idx(0, 0)

            def body(i3, carry):
                for b in range(3):
                    i = 3 * i3 + b
                    wait_idx(b)
                    pltpu.async_copy(y_hbm.at[ch].at[si[b]], rows[b], gsem[b])

                    @pl.when(i >= 1)
                    def _():
                        retire_gather((b + 2) % 3, ch)

                    @pl.when(i >= 2)
                    def _():
                        wait_scatter((b + 1) % 3)

                    @pl.when(i + 1 < NBATCH)
                    def _():
                        issue_idx(i + 1, (b + 1) % 3)
                return carry

            lax.fori_loop(0, NBATCH // 3, body, 0)
            retire_gather((NBATCH - 1) % 3, ch)
            wait_scatter((NBATCH - 2) % 3)
            wait_scatter((NBATCH - 1) % 3)
            plsc.subcore_barrier()
            pltpu.sync_copy(buf.at[pl.ds(sid * TROWS, TROWS)],
                            out_hbm.at[ch].at[pl.ds(sid * TROWS, TROWS)])

    return k(y, srcr, dstr)


def _affine_from_stats(st_ref, g_ref, be_ref):
    m = st_ref[0:1, :] * (1.0 / N)
    var = st_ref[1:2, :] * (1.0 / N) - m * m
    a = g_ref[...] * lax.rsqrt(var + EPS)
    c = be_ref[...] - m * a
    return a, c


def _acc_stats(st_ref, o, first):
    @pl.when(first)
    def _():
        st_ref[...] = jnp.zeros_like(st_ref)
    st_ref[0:1, :] += jnp.sum(o, axis=0, keepdims=True)
    st_ref[1:2, :] += jnp.sum(o * o, axis=0, keepdims=True)


def _mm_stats_body(x_ref, w_ref, b_ref, out_ref, st_ref):
    o = jnp.dot(x_ref[...], w_ref[...],
                preferred_element_type=jnp.float32) + b_ref[...]
    out_ref[...] = o
    _acc_stats(st_ref, o, pl.program_id(0) == 0)


def _mm_stats(x, w, b):
    return pl.pallas_call(
        _mm_stats_body,
        grid=(NR,),
        in_specs=[pl.BlockSpec((R, D), lambda i: (i, 0)),
                  pl.BlockSpec((D, D), lambda i: (0, 0)),
                  pl.BlockSpec((1, D), lambda i: (0, 0))],
        out_specs=[pl.BlockSpec((R, D), lambda i: (i, 0)),
                   pl.BlockSpec((2, D), lambda i: (0, 0))],
        out_shape=[jax.ShapeDtypeStruct((N, D), jnp.float32),
                   jax.ShapeDtypeStruct((2, D), jnp.float32)],
    )(x, w, b)


def _store_chunked(y_ref, yfull):
    for c in range(NCH):
        y_ref[c, :, :] = yfull[:, c * CW:(c + 1) * CW]


def _conv1_body(u_ref, st_ref, g_ref, be_ref, w_ref, dinv_ref, y_ref, h_ref):
    a, c = _affine_from_stats(st_ref, g_ref, be_ref)
    h = jnp.maximum(u_ref[...] * a + c, 0.0)
    h_ref[...] = h
    _store_chunked(y_ref, jnp.dot(h, w_ref[...],
                                  preferred_element_type=jnp.float32)
                   * dinv_ref[...])


def _conv1(u, st, g, be, w, dinv):
    return pl.pallas_call(
        _conv1_body,
        grid=(NR,),
        in_specs=[pl.BlockSpec((R, D), lambda i: (i, 0)),
                  pl.BlockSpec((2, D), lambda i: (0, 0)),
                  pl.BlockSpec((1, D), lambda i: (0, 0)),
                  pl.BlockSpec((1, D), lambda i: (0, 0)),
                  pl.BlockSpec((D, D), lambda i: (0, 0)),
                  pl.BlockSpec((R, 1), lambda i: (i, 0))],
        out_specs=[pl.BlockSpec((NCH, R, CW), lambda i: (0, i, 0)),
                   pl.BlockSpec((R, D), lambda i: (i, 0))],
        out_shape=[jax.ShapeDtypeStruct((NCH, NP, CW), jnp.float32),
                   jax.ShapeDtypeStruct((N, D), jnp.float32)],
    )(u, st, g, be, w, dinv)


def _combine(agg_ref, hp_ref, bc_ref, dinv_ref):
    agg = jnp.concatenate([agg_ref[kk] for kk in range(NCH)], axis=1)
    return jnp.maximum(agg * dinv_ref[...] + bc_ref[...] + hp_ref[...], 0.0)


def _conv23_body(agg_ref, hp_ref, bc_ref, dinv_ref, w_ref, y_ref, h_ref):
    x = _combine(agg_ref, hp_ref, bc_ref, dinv_ref)
    h_ref[...] = x
    _store_chunked(y_ref, jnp.dot(x, w_ref[...],
                                  preferred_element_type=jnp.float32)
                   * dinv_ref[...])


def _conv23(agg, hp, bc, dinv, w):
    return pl.pallas_call(
        _conv23_body,
        grid=(NR,),
        in_specs=[pl.BlockSpec((NCH, R, CW), lambda i: (0, i, 0)),
                  pl.BlockSpec((R, D), lambda i: (i, 0)),
                  pl.BlockSpec((1, D), lambda i: (0, 0)),
                  pl.BlockSpec((R, 1), lambda i: (i, 0)),
                  pl.BlockSpec((D, D), lambda i: (0, 0))],
        out_specs=[pl.BlockSpec((NCH, R, CW), lambda i: (0, i, 0)),
                   pl.BlockSpec((R, D), lambda i: (i, 0))],
        out_shape=[jax.ShapeDtypeStruct((NCH, NP, CW), jnp.float32),
                   jax.ShapeDtypeStruct((N, D), jnp.float32)],
    )(agg, hp, bc, dinv, w)


def _postA_body(agg_ref, hp_ref, bc_ref, dinv_ref, w_ref, b_ref, v_ref, st_ref):
    x = _combine(agg_ref, hp_ref, bc_ref, dinv_ref)
    v = jnp.dot(x, w_ref[...], preferred_element_type=jnp.float32) + b_ref[...]
    v_ref[...] = v
    _acc_stats(st_ref, v, pl.program_id(0) == 0)


def _postA(agg, hp, bc, dinv, w, b):
    return pl.pallas_call(
        _postA_body,
        grid=(NR,),
        in_specs=[pl.BlockSpec((NCH, R, CW), lambda i: (0, i, 0)),
                  pl.BlockSpec((R, D), lambda i: (i, 0)),
                  pl.BlockSpec((1, D), lambda i: (0, 0)),
                  pl.BlockSpec((R, 1), lambda i: (i, 0)),
                  pl.BlockSpec((D, D), lambda i: (0, 0)),
                  pl.BlockSpec((1, D), lambda i: (0, 0))],
        out_specs=[pl.BlockSpec((R, D), lambda i: (i, 0)),
                   pl.BlockSpec((2, D), lambda i: (0, 0))],
        out_shape=[jax.ShapeDtypeStruct((N, D), jnp.float32),
                   jax.ShapeDtypeStruct((2, D), jnp.float32)],
    )(agg, hp, bc, dinv, w, b)


def _postB_body(u_ref, pst_ref, g_ref, be_ref, w_ref, b_ref, v_ref, st_ref):
    a, c = _affine_from_stats(pst_ref, g_ref, be_ref)
    x = jnp.maximum(u_ref[...] * a + c, 0.0)
    v = jnp.dot(x, w_ref[...], preferred_element_type=jnp.float32) + b_ref[...]
    v_ref[...] = v
    _acc_stats(st_ref, v, pl.program_id(0) == 0)


def _postB(u, pst, g, be, w, b):
    return pl.pallas_call(
        _postB_body,
        grid=(NR,),
        in_specs=[pl.BlockSpec((R, D), lambda i: (i, 0)),
                  pl.BlockSpec((2, D), lambda i: (0, 0)),
                  pl.BlockSpec((1, D), lambda i: (0, 0)),
                  pl.BlockSpec((1, D), lambda i: (0, 0)),
                  pl.BlockSpec((D, D), lambda i: (0, 0)),
                  pl.BlockSpec((1, D), lambda i: (0, 0))],
        out_specs=[pl.BlockSpec((R, D), lambda i: (i, 0)),
                   pl.BlockSpec((2, D), lambda i: (0, 0))],
        out_shape=[jax.ShapeDtypeStruct((N, D), jnp.float32),
                   jax.ShapeDtypeStruct((2, D), jnp.float32)],
    )(u, pst, g, be, w, b)


def _pool_body(v_ref, pst_ref, g_ref, be_ref, bt_ref, s_ref, c_ref):
    a, c0 = _affine_from_stats(pst_ref, g_ref, be_ref)
    xn = v_ref[...] * a + c0
    oh = (bt_ref[...] == lax.broadcasted_iota(jnp.int32, (1, NB), 1))
    oh = oh.astype(jnp.float32)

    @pl.when(pl.program_id(0) == 0)
    def _():
        s_ref[...] = jnp.zeros_like(s_ref)
        c_ref[...] = jnp.zeros_like(c_ref)

    dn = (((0,), (0,)), ((), ()))
    s_ref[...] += lax.dot_general(oh, xn, dn,
                                  preferred_element_type=jnp.float32)
    c_ref[...] += lax.dot_general(oh, jnp.ones_like(xn), dn,
                                  preferred_element_type=jnp.float32)


def _pool(v, pst, g, be, bt):
    return pl.pallas_call(
        _pool_body,
        grid=(NR,),
        in_specs=[pl.BlockSpec((R, D), lambda i: (i, 0)),
                  pl.BlockSpec((2, D), lambda i: (0, 0)),
                  pl.BlockSpec((1, D), lambda i: (0, 0)),
                  pl.BlockSpec((1, D), lambda i: (0, 0)),
                  pl.BlockSpec((R, 1), lambda i: (i, 0))],
        out_specs=[pl.BlockSpec((NB, D), lambda i: (0, 0)),
                   pl.BlockSpec((NB, D), lambda i: (0, 0))],
        out_shape=[jax.ShapeDtypeStruct((NB, D), jnp.float32),
                   jax.ShapeDtypeStruct((NB, D), jnp.float32)],
    )(v, pst, g, be, bt)


def _final_body(s1_ref, c1_ref, s2_ref, c2_ref, w1a_ref, w1b_ref, b1_ref,
                w2_ref, b2_ref, w3_ref, b3_ref, out_ref):
    m1 = s1_ref[...] / jnp.maximum(c1_ref[...], 1.0)
    m2 = s2_ref[...] / jnp.maximum(c2_ref[...], 1.0)
    g = jnp.dot(m1, w1a_ref[...], preferred_element_type=jnp.float32)
    g += jnp.dot(m2, w1b_ref[...], preferred_element_type=jnp.float32)
    g = jnp.maximum(g + b1_ref[...], 0.0)
    g = jnp.maximum(jnp.dot(g, w2_ref[...],
                            preferred_element_type=jnp.float32) + b2_ref[...], 0.0)
    out_ref[...] = jnp.dot(g, w3_ref[...],
                           preferred_element_type=jnp.float32) + b3_ref[...]


def _final(s1, c1, s2, c2, w1a, w1b, b1, w2, b2, w3p, b3p):
    return pl.pallas_call(
        _final_body,
        out_shape=jax.ShapeDtypeStruct((NB, D), jnp.float32),
    )(s1, c1, s2, c2, w1a, w1b, b1, w2, b2, w3p, b3p)


def _dinv_body(dg_ref, out_ref):
    out_ref[...] = lax.rsqrt(dg_ref[:, 0:1] + 1.0)


def _dinv(deg4):
    return pl.pallas_call(
        _dinv_body,
        grid=(NR,),
        in_specs=[pl.BlockSpec((R, DEGW), lambda i: (i, 0))],
        out_specs=pl.BlockSpec((R, 1), lambda i: (i, 0)),
        out_shape=jax.ShapeDtypeStruct((N, 1), jnp.float32),
    )(deg4)


def kernel(x_graph_1, x_graph_2, edge_index_g1, edge_index_g2, batch_g1,
           batch_g2, pre_W, pre_b, pre_gamma, pre_beta, conv_W, conv_b,
           post_W, post_b, post_gamma, post_beta, fin_W1, fin_b1, fin_W2,
           fin_b2, fin_W3, fin_b3):
    zrows = jnp.zeros((NP, DEGW), jnp.float32)
    orows = jnp.ones((B, DEGW), jnp.float32)
    pad = jnp.full((EPAD - E,), N, jnp.int32)

    pooled = []
    for j, (x, ei, bt) in enumerate(((x_graph_1, edge_index_g1, batch_g1),
                                     (x_graph_2, edge_index_g2, batch_g2))):
        srcr = jnp.concatenate([ei[0], pad])
        dstr = jnp.concatenate([ei[1], pad])

        deg4 = _sc_degree(dstr, zrows, orows)
        dinv = _dinv(deg4)

        u0, st0 = _mm_stats(x, pre_W[2, j], pre_b[2, j][None])

        y, h = _conv1(u0, st0, pre_gamma[2, j][None], pre_beta[2, j][None],
                      conv_W[0, j], dinv)
        agg = _sc_scatter(y, srcr, dstr)
        y, h = _conv23(agg, h, conv_b[0, j][None], dinv, conv_W[1, j])
        agg = _sc_scatter(y, srcr, dstr)
        y, h = _conv23(agg, h, conv_b[1, j][None], dinv, conv_W[2, j])
        agg = _sc_scatter(y, srcr, dstr)

        v, st = _postA(agg, h, conv_b[2, j][None], dinv,
                       post_W[0, j], post_b[0, j][None])
        v, st = _postB(v, st, post_gamma[0, j][None], post_beta[0, j][None],
                       post_W[1, j], post_b[1, j][None])
        v, st = _postB(v, st, post_gamma[1, j][None], post_beta[1, j][None],
                       post_W[2, j], post_b[2, j][None])
        s, c = _pool(v, st, post_gamma[2, j][None], post_beta[2, j][None],
                     bt.reshape(N, 1))
        pooled.append((s, c))

    w3p = jnp.pad(fin_W3, ((0, 0), (0, D - OUT)))
    b3p = jnp.pad(fin_b3, (0, D - OUT))[None]
    out = _final(pooled[0][0], pooled[0][1], pooled[1][0], pooled[1][1],
                 fin_W1[:D], fin_W1[D:], fin_b1[None], fin_W2, fin_b2[None],
                 w3p, b3p)
    return out[:, :OUT]


# speedup vs baseline: 1.5075x; 1.0722x over previous
import functools

import jax
import jax.numpy as jnp
from jax import lax
from jax.experimental import pallas as pl
from jax.experimental.pallas import tpu as pltpu
from jax.experimental.pallas import tpu_sc as plsc

N = 50000
E = 800000
D = 128
NB = 16
OUT = 7
EPS = 1e-5

R = 2000
NR = N // R
NCH = 4
CW = D // NCH
NP = N + 48
B = 296
EB = 50616
EPAD = 16 * EB
NBATCH = EB // B
TROWS = NP // 16
DEGW = 16

def _mesh():
    return plsc.VectorSubcoreMesh(core_axis_name="c", subcore_axis_name="s")


def _sc_degree(dstr, zrows, orows):
    @functools.partial(
        pl.kernel,
        mesh=_mesh(),
        compiler_params=pltpu.CompilerParams(use_tc_tiling_on_sc=False),
        out_type=jax.ShapeDtypeStruct((NP, DEGW), jnp.float32),
        scratch_types=[
            pltpu.VMEM((B,), jnp.int32),
            pltpu.VMEM((B, DEGW), jnp.float32),
            pltpu.VMEM_SHARED((NP, DEGW), jnp.float32),
        ],
    )
    def k(dst_hbm, z_hbm, one_hbm, out_hbm, di, ones_v, buf):
        cid = lax.axis_index("c")
        sid = lax.axis_index("s")

        @pl.when(cid == 0)
        def _():
            pltpu.sync_copy(z_hbm.at[pl.ds(sid * TROWS, TROWS)],
                            buf.at[pl.ds(sid * TROWS, TROWS)])
            pltpu.sync_copy(one_hbm, ones_v)
            plsc.subcore_barrier()

            def body(i, carry):
                e0 = sid * EB + i * B
                pltpu.sync_copy(dst_hbm.at[pl.ds(e0, B)], di)
                pltpu.sync_copy(ones_v, buf.at[di], add=True)
                return carry

            lax.fori_loop(0, NBATCH, body, 0)
            plsc.subcore_barrier()
            pltpu.sync_copy(buf.at[pl.ds(sid * TROWS, TROWS)],
                            out_hbm.at[pl.ds(sid * TROWS, TROWS)])

    return k(dstr, zrows, orows)


def _sc_scatter(y, srcr, dstr):
    @functools.partial(
        pl.kernel,
        mesh=_mesh(),
        compiler_params=pltpu.CompilerParams(use_tc_tiling_on_sc=False),
        out_type=jax.ShapeDtypeStruct((NCH, NP, CW), jnp.float32),
        scratch_types=[
            [pltpu.VMEM((B,), jnp.int32)] * 3,
            [pltpu.VMEM((B,), jnp.int32)] * 3,
            [pltpu.VMEM((B, CW), jnp.float32)] * 3,
            pltpu.VMEM_SHARED((NP, CW), jnp.float32),
            [pltpu.SemaphoreType.DMA] * 3,
            [pltpu.SemaphoreType.DMA] * 3,
            [pltpu.SemaphoreType.DMA] * 3,
        ],
    )
    def k(y_hbm, src_hbm, dst_hbm, out_hbm, si, di, rows, buf, isem, gsem,
          ssem):
        cid = lax.axis_index("c")
        sid = lax.axis_index("s")

        def issue_idx(i, b):
            e0 = sid * EB + i * B
            pltpu.async_copy(src_hbm.at[pl.ds(e0, B)], si[b], isem[b])
            pltpu.async_copy(dst_hbm.at[pl.ds(e0, B)], di[b], isem[b])

        def wait_idx(b):
            pltpu.make_async_copy(src_hbm.at[pl.ds(0, B)], si[b],
                                  isem[b]).wait()
            pltpu.make_async_copy(dst_hbm.at[pl.ds(0, B)], di[b],
                                  isem[b]).wait()

        def wait_scatter(b):
            pltpu.make_async_copy(rows[b], buf.at[di[b]], ssem[b]).wait()

        def retire_gather(b, ch):
            pltpu.make_async_copy(y_hbm.at[ch].at[si[b]], rows[b],
                                  gsem[b]).wait()
            pltpu.async_copy(rows[b], buf.at[di[b]], ssem[b], add=True)

        for kk in range(NCH // 2):
            ch = cid * (NCH // 2) + kk
            pltpu.sync_copy(y_hbm.at[ch].at[pl.ds(sid * TROWS, TROWS)],
                            buf.at[pl.ds(sid * TROWS, TROWS)])
            plsc.subcore_barrier()

            issue_idx(0, 0)

            def body(i3, carry):
                for b in range(3):
                    i = 3 * i3 + b
                    wait_idx(b)
                    pltpu.async_copy(y_hbm.at[ch].at[si[b]], rows[b], gsem[b])

                    @pl.when(i >= 1)
                    def _():
                        retire_gather((b + 2) % 3, ch)

                    @pl.when(i >= 2)
                    def _():
                        wait_scatter((b + 1) % 3)

                    @pl.when(i + 1 < NBATCH)
                    def _():
                        issue_idx(i + 1, (b + 1) % 3)
                return carry

            lax.fori_loop(0, NBATCH // 3, body, 0)
            retire_gather((NBATCH - 1) % 3, ch)
            wait_scatter((NBATCH - 2) % 3)
            wait_scatter((NBATCH - 1) % 3)
            plsc.subcore_barrier()
            pltpu.sync_copy(buf.at[pl.ds(sid * TROWS, TROWS)],
                            out_hbm.at[ch].at[pl.ds(sid * TROWS, TROWS)])

    return k(y, srcr, dstr)


def _affine_from_stats(st_ref, g_ref, be_ref):
    m = st_ref[0:1, :] * (1.0 / N)
    var = st_ref[1:2, :] * (1.0 / N) - m * m
    a = g_ref[...] * lax.rsqrt(var + EPS)
    c = be_ref[...] - m * a
    return a, c


def _acc_stats(st_ref, o, first):
    @pl.when(first)
    def _():
        st_ref[...] = jnp.zeros_like(st_ref)
    st_ref[0:1, :] += jnp.sum(o, axis=0, keepdims=True)
    st_ref[1:2, :] += jnp.sum(o * o, axis=0, keepdims=True)


def _mm_stats_body(x_ref, w_ref, b_ref, out_ref, st_ref):
    o = jnp.dot(x_ref[...], w_ref[...],
                preferred_element_type=jnp.float32) + b_ref[...]
    out_ref[...] = o
    _acc_stats(st_ref, o, pl.program_id(0) == 0)


def _mm_stats(x, w, b):
    return pl.pallas_call(
        _mm_stats_body,
        grid=(NR,),
        in_specs=[pl.BlockSpec((R, D), lambda i: (i, 0)),
                  pl.BlockSpec((D, D), lambda i: (0, 0)),
                  pl.BlockSpec((1, D), lambda i: (0, 0))],
        out_specs=[pl.BlockSpec((R, D), lambda i: (i, 0)),
                   pl.BlockSpec((2, D), lambda i: (0, 0))],
        out_shape=[jax.ShapeDtypeStruct((N, D), jnp.float32),
                   jax.ShapeDtypeStruct((2, D), jnp.float32)],
    )(x, w, b)


def _store_chunked(y_ref, yfull):
    for c in range(NCH):
        y_ref[c, :, :] = yfull[:, c * CW:(c + 1) * CW]


def _conv1_body(u_ref, st_ref, g_ref, be_ref, w_ref, dinv_ref, y_ref, h_ref):
    a, c = _affine_from_stats(st_ref, g_ref, be_ref)
    h = jnp.maximum(u_ref[...] * a + c, 0.0)
    h_ref[...] = h
    _store_chunked(y_ref, jnp.dot(h, w_ref[...],
                                  preferred_element_type=jnp.float32)
                   * dinv_ref[...])


def _conv1(u, st, g, be, w, dinv):
    return pl.pallas_call(
        _conv1_body,
        grid=(NR,),
        in_specs=[pl.BlockSpec((R, D), lambda i: (i, 0)),
                  pl.BlockSpec((2, D), lambda i: (0, 0)),
                  pl.BlockSpec((1, D), lambda i: (0, 0)),
                  pl.BlockSpec((1, D), lambda i: (0, 0)),
                  pl.BlockSpec((D, D), lambda i: (0, 0)),
                  pl.BlockSpec((R, 1), lambda i: (i, 0))],
        out_specs=[pl.BlockSpec((NCH, R, CW), lambda i: (0, i, 0)),
                   pl.BlockSpec((R, D), lambda i: (i, 0))],
        out_shape=[jax.ShapeDtypeStruct((NCH, NP, CW), jnp.float32),
                   jax.ShapeDtypeStruct((N, D), jnp.float32)],
    )(u, st, g, be, w, dinv)


def _combine(agg_ref, hp_ref, bc_ref, dinv_ref):
    agg = jnp.concatenate([agg_ref[kk] for kk in range(NCH)], axis=1)
    return jnp.maximum(agg * dinv_ref[...] + bc_ref[...] + hp_ref[...], 0.0)


def _conv23_body(agg_ref, hp_ref, bc_ref, dinv_ref, w_ref, y_ref, h_ref):
    x = _combine(agg_ref, hp_ref, bc_ref, dinv_ref)
    h_ref[...] = x
    _store_chunked(y_ref, jnp.dot(x, w_ref[...],
                                  preferred_element_type=jnp.float32)
                   * dinv_ref[...])


def _conv23(agg, hp, bc, dinv, w):
    return pl.pallas_call(
        _conv23_body,
        grid=(NR,),
        in_specs=[pl.BlockSpec((NCH, R, CW), lambda i: (0, i, 0)),
                  pl.BlockSpec((R, D), lambda i: (i, 0)),
                  pl.BlockSpec((1, D), lambda i: (0, 0)),
                  pl.BlockSpec((R, 1), lambda i: (i, 0)),
                  pl.BlockSpec((D, D), lambda i: (0, 0))],
        out_specs=[pl.BlockSpec((NCH, R, CW), lambda i: (0, i, 0)),
                   pl.BlockSpec((R, D), lambda i: (i, 0))],
        out_shape=[jax.ShapeDtypeStruct((NCH, NP, CW), jnp.float32),
                   jax.ShapeDtypeStruct((N, D), jnp.float32)],
    )(agg, hp, bc, dinv, w)


def _postA_body(agg_ref, hp_ref, bc_ref, dinv_ref, w_ref, b_ref, v_ref, st_ref):
    x = _combine(agg_ref, hp_ref, bc_ref, dinv_ref)
    v = jnp.dot(x, w_ref[...], preferred_element_type=jnp.float32) + b_ref[...]
    v_ref[...] = v
    _acc_stats(st_ref, v, pl.program_id(0) == 0)


def _postA(agg, hp, bc, dinv, w, b):
    return pl.pallas_call(
        _postA_body,
        grid=(NR,),
        in_specs=[pl.BlockSpec((NCH, R, CW), lambda i: (0, i, 0)),
                  pl.BlockSpec((R, D), lambda i: (i, 0)),
                  pl.BlockSpec((1, D), lambda i: (0, 0)),
                  pl.BlockSpec((R, 1), lambda i: (i, 0)),
                  pl.BlockSpec((D, D), lambda i: (0, 0)),
                  pl.BlockSpec((1, D), lambda i: (0, 0))],
        out_specs=[pl.BlockSpec((R, D), lambda i: (i, 0)),
                   pl.BlockSpec((2, D), lambda i: (0, 0))],
        out_shape=[jax.ShapeDtypeStruct((N, D), jnp.float32),
                   jax.ShapeDtypeStruct((2, D), jnp.float32)],
    )(agg, hp, bc, dinv, w, b)


def _postB_body(u_ref, pst_ref, g_ref, be_ref, w_ref, b_ref, v_ref, st_ref):
    a, c = _affine_from_stats(pst_ref, g_ref, be_ref)
    x = jnp.maximum(u_ref[...] * a + c, 0.0)
    v = jnp.dot(x, w_ref[...], preferred_element_type=jnp.float32) + b_ref[...]
    v_ref[...] = v
    _acc_stats(st_ref, v, pl.program_id(0) == 0)


def _postB(u, pst, g, be, w, b):
    return pl.pallas_call(
        _postB_body,
        grid=(NR,),
        in_specs=[pl.BlockSpec((R, D), lambda i: (i, 0)),
                  pl.BlockSpec((2, D), lambda i: (0, 0)),
                  pl.BlockSpec((1, D), lambda i: (0, 0)),
                  pl.BlockSpec((1, D), lambda i: (0, 0)),
                  pl.BlockSpec((D, D), lambda i: (0, 0)),
                  pl.BlockSpec((1, D), lambda i: (0, 0))],
        out_specs=[pl.BlockSpec((R, D), lambda i: (i, 0)),
                   pl.BlockSpec((2, D), lambda i: (0, 0))],
        out_shape=[jax.ShapeDtypeStruct((N, D), jnp.float32),
                   jax.ShapeDtypeStruct((2, D), jnp.float32)],
    )(u, pst, g, be, w, b)


def _pool_body(v_ref, pst_ref, g_ref, be_ref, bt_ref, s_ref, c_ref):
    a, c0 = _affine_from_stats(pst_ref, g_ref, be_ref)
    xn = v_ref[...] * a + c0
    oh = (bt_ref[...] == lax.broadcasted_iota(jnp.int32, (1, NB), 1))
    oh = oh.astype(jnp.float32)

    @pl.when(pl.program_id(0) == 0)
    def _():
        s_ref[...] = jnp.zeros_like(s_ref)
        c_ref[...] = jnp.zeros_like(c_ref)

    dn = (((0,), (0,)), ((), ()))
    s_ref[...] += lax.dot_general(oh, xn, dn,
                                  preferred_element_type=jnp.float32)
    c_ref[...] += lax.dot_general(oh, jnp.ones_like(xn), dn,
                                  preferred_element_type=jnp.float32)


def _pool(v, pst, g, be, bt):
    return pl.pallas_call(
        _pool_body,
        grid=(NR,),
        in_specs=[pl.BlockSpec((R, D), lambda i: (i, 0)),
                  pl.BlockSpec((2, D), lambda i: (0, 0)),
                  pl.BlockSpec((1, D), lambda i: (0, 0)),
                  pl.BlockSpec((1, D), lambda i: (0, 0)),
                  pl.BlockSpec((R, 1), lambda i: (i, 0))],
        out_specs=[pl.BlockSpec((NB, D), lambda i: (0, 0)),
                   pl.BlockSpec((NB, D), lambda i: (0, 0))],
        out_shape=[jax.ShapeDtypeStruct((NB, D), jnp.float32),
                   jax.ShapeDtypeStruct((NB, D), jnp.float32)],
    )(v, pst, g, be, bt)


def _final_body(s1_ref, c1_ref, s2_ref, c2_ref, w1a_ref, w1b_ref, b1_ref,
                w2_ref, b2_ref, w3_ref, b3_ref, out_ref):
    m1 = s1_ref[...] / jnp.maximum(c1_ref[...], 1.0)
    m2 = s2_ref[...] / jnp.maximum(c2_ref[...], 1.0)
    g = jnp.dot(m1, w1a_ref[...], preferred_element_type=jnp.float32)
    g += jnp.dot(m2, w1b_ref[...], preferred_element_type=jnp.float32)
    g = jnp.maximum(g + b1_ref[...], 0.0)
    g = jnp.maximum(jnp.dot(g, w2_ref[...],
                            preferred_element_type=jnp.float32) + b2_ref[...], 0.0)
    out_ref[...] = jnp.dot(g, w3_ref[...],
                           preferred_element_type=jnp.float32) + b3_ref[...]


def _final(s1, c1, s2, c2, w1a, w1b, b1, w2, b2, w3p, b3p):
    return pl.pallas_call(
        _final_body,
        out_shape=jax.ShapeDtypeStruct((NB, D), jnp.float32),
    )(s1, c1, s2, c2, w1a, w1b, b1, w2, b2, w3p, b3p)


def _dinv_body(dg_ref, out_ref):
    out_ref[...] = lax.rsqrt(dg_ref[:, 0:1] + 1.0)


def _dinv(deg4):
    return pl.pallas_call(
        _dinv_body,
        grid=(NR,),
        in_specs=[pl.BlockSpec((R, DEGW), lambda i: (i, 0))],
        out_specs=pl.BlockSpec((R, 1), lambda i: (i, 0)),
        out_shape=jax.ShapeDtypeStruct((N, 1), jnp.float32),
    )(deg4)


def kernel(x_graph_1, x_graph_2, edge_index_g1, edge_index_g2, batch_g1,
           batch_g2, pre_W, pre_b, pre_gamma, pre_beta, conv_W, conv_b,
           post_W, post_b, post_gamma, post_beta, fin_W1, fin_b1, fin_W2,
           fin_b2, fin_W3, fin_b3):
    zrows = jnp.zeros((NP, DEGW), jnp.float32)
    orows = jnp.ones((B, DEGW), jnp.float32)
    pad = jnp.full((EPAD - E,), N, jnp.int32)

    pooled = []
    for j, (x, ei, bt) in enumerate(((x_graph_1, edge_index_g1, batch_g1),
                                     (x_graph_2, edge_index_g2, batch_g2))):
        srcr = jnp.concatenate([ei[0], pad])
        dstr = jnp.concatenate([ei[1], pad])

        deg4 = _sc_degree(dstr, zrows, orows)
        dinv = _dinv(deg4)

        u0, st0 = _mm_stats(x, pre_W[2, j], pre_b[2, j][None])

        y, h = _conv1(u0, st0, pre_gamma[2, j][None], pre_beta[2, j][None],
                      conv_W[0, j], dinv)
        agg = _sc_scatter(y, srcr, dstr)
        y, h = _conv23(agg, h, conv_b[0, j][None], dinv, conv_W[1, j])
        agg = _sc_scatter(y, srcr, dstr)
        y, h = _conv23(agg, h, conv_b[1, j][None], dinv, conv_W[2, j])
        agg = _sc_scatter(y, srcr, dstr)

        v, st = _postA(agg, h, conv_b[2, j][None], dinv,
                       post_W[0, j], post_b[0, j][None])
        v, st = _postB(v, st, post_gamma[0, j][None], post_beta[0, j][None],
                       post_W[1, j], post_b[1, j][None])
        v, st = _postB(v, st, post_gamma[1, j][None], post_beta[1, j][None],
                       post_W[2, j], post_b[2, j][None])
        s, c = _pool(v, st, post_gamma[2, j][None], post_beta[2, j][None],
                     bt.reshape(N, 1))
        pooled.append((s, c))

    w3p = jnp.pad(fin_W3, ((0, 0), (0, D - OUT)))
    b3p = jnp.pad(fin_b3, (0, D - OUT))[None]
    out = _final(pooled[0][0], pooled[0][1], pooled[1][0], pooled[1][1],
                 fin_W1[:D], fin_W1[D:], fin_b1[None], fin_W2, fin_b2[None],
                 w3p, b3p)
    return out[:, :OUT]
